# baseline (speedup 1.0000x reference)
import dataclasses
import functools

import numpy as np
import jax
import jax.numpy as jnp
from jax import lax
from jax.experimental import pallas as pl
from jax.experimental.pallas import tpu as pltpu
from jax.experimental.pallas import tpu_sc as plsc

N_ATOMS = 100000
N_COLS = 6
LANES = 16
NUM_WORKERS = 32

N_TILES = 782
TPW = 25
LAST_TPW = N_TILES - 31 * TPW
Z_VALS = (6, 7, 8, 15, 16)


def _compute_tile(a_ref, o_ref, t):
    one = jnp.full((LANES,), 1.0, jnp.float32)
    zero = jnp.zeros((LANES,), jnp.float32)
    for l in range(8):
        a16 = a_ref[pl.ds(t * 128 + l * LANES, LANES)]
        vals = [jnp.where(a16 == z, one, zero) for z in Z_VALS]
        v5 = one - (vals[0] + vals[1] + vals[2] + vals[3] + vals[4])
        vals.append(v5)
        for j in range(N_COLS):
            o_ref[t, j, pl.ds(l * LANES, LANES)] = vals[j]


def _worker_body(a_ref, o_ref, ntiles):
    @pl.loop(0, ntiles)
    def _(t):
        _compute_tile(a_ref, o_ref, t)


SLAB = 5


def _sc_kernel(a_hbm, out_hbm, a_v, o_v, sem):
    wid = lax.axis_index("s") * 2 + lax.axis_index("c")

    @pl.when(wid < 31)
    def _():
        base = wid * (TPW * 128)
        pltpu.sync_copy(a_hbm.at[pl.ds(base, TPW * 128)], a_v)
        copies = []
        for s in range(TPW // SLAB):
            @pl.loop(0, SLAB)
            def _(t, s=s):
                _compute_tile(a_v, o_v, s * SLAB + t)

            copies.append(
                pltpu.async_copy(
                    o_v.at[pl.ds(s * SLAB, SLAB)],
                    out_hbm.at[pl.ds(wid * TPW + s * SLAB, SLAB)],
                    sem,
                )
            )
        for c in copies:
            c.wait()

    @pl.when(wid == 31)
    def _():
        base = 31 * (TPW * 128)
        pltpu.sync_copy(
            a_hbm.at[pl.ds(base, N_ATOMS - base)],
            a_v.at[pl.ds(0, N_ATOMS - base)],
        )
        _worker_body(a_v, o_v, LAST_TPW)
        pltpu.sync_copy(
            o_v.at[pl.ds(0, LAST_TPW)], out_hbm.at[pl.ds(31 * TPW, LAST_TPW)]
        )


def _compiler_params():
    cp = pltpu.CompilerParams()
    if "needs_layout_passes" in pltpu.CompilerParams.__dataclass_fields__:
        cp = dataclasses.replace(cp, needs_layout_passes=False)
    return cp


@jax.jit
def _embed(atomic_numbers):
    mesh = plsc.VectorSubcoreMesh(core_axis_name="c", subcore_axis_name="s")
    run = pl.kernel(
        _sc_kernel,
        out_type=jax.ShapeDtypeStruct((N_TILES, 8, 128), jnp.float32),
        mesh=mesh,
        compiler_params=_compiler_params(),
        scratch_types=[
            pltpu.VMEM((TPW * 128,), jnp.int32),
            pltpu.VMEM((TPW, 8, 128), jnp.float32),
            pltpu.SemaphoreType.DMA,
        ],
    )
    t = run(atomic_numbers)
    return t.transpose(1, 0, 2).reshape(8, N_TILES * 128)[:N_COLS, :N_ATOMS].T


def kernel(atomic_numbers):
    return _embed(atomic_numbers)


# speedup vs baseline: 1.0710x; 1.0710x over previous
import dataclasses
import functools

import numpy as np
import jax
import jax.numpy as jnp
from jax import lax
from jax.experimental import pallas as pl
from jax.experimental.pallas import tpu as pltpu
from jax.experimental.pallas import tpu_sc as plsc

N_ATOMS = 100000
N_COLS = 6
LANES = 16
NUM_WORKERS = 32

N_TILES = 782
TPW = 25
LAST_TPW = N_TILES - 31 * TPW
Z_VALS = (6, 7, 8, 15, 16)


def _compute_tile(a_ref, o_ref, t):
    one = jnp.full((LANES,), 1.0, jnp.float32)
    zero = jnp.zeros((LANES,), jnp.float32)
    for l in range(8):
        a16 = a_ref[pl.ds(t * 128 + l * LANES, LANES)]
        vals = [jnp.where(a16 == z, one, zero) for z in Z_VALS]
        v5 = one - (vals[0] + vals[1] + vals[2] + vals[3] + vals[4])
        vals.append(v5)
        for j in range(N_COLS):
            o_ref[t, j, pl.ds(l * LANES, LANES)] = vals[j]


def _worker_body(a_ref, o_ref, ntiles):
    @pl.loop(0, ntiles)
    def _(t):
        _compute_tile(a_ref, o_ref, t)


def _sc_kernel(a_hbm, out_hbm, a_v, o_v, sem):
    wid = lax.axis_index("s") * 2 + lax.axis_index("c")

    @pl.when(wid < 31)
    def _():
        base = wid * (TPW * 128)
        pltpu.sync_copy(a_hbm.at[pl.ds(base, TPW * 128)], a_v)
        _worker_body(a_v, o_v, TPW)
        pltpu.sync_copy(o_v, out_hbm.at[pl.ds(wid * TPW, TPW)])

    @pl.when(wid == 31)
    def _():
        base = 31 * (TPW * 128)
        pltpu.sync_copy(
            a_hbm.at[pl.ds(base, N_ATOMS - base)],
            a_v.at[pl.ds(0, N_ATOMS - base)],
        )
        _worker_body(a_v, o_v, LAST_TPW)
        pltpu.sync_copy(
            o_v.at[pl.ds(0, LAST_TPW)], out_hbm.at[pl.ds(31 * TPW, LAST_TPW)]
        )


def _compiler_params():
    cp = pltpu.CompilerParams()
    if "needs_layout_passes" in pltpu.CompilerParams.__dataclass_fields__:
        cp = dataclasses.replace(cp, needs_layout_passes=False)
    return cp


@jax.jit
def _embed(atomic_numbers):
    mesh = plsc.VectorSubcoreMesh(core_axis_name="c", subcore_axis_name="s")
    run = pl.kernel(
        _sc_kernel,
        out_type=jax.ShapeDtypeStruct((N_TILES, 8, 128), jnp.float32),
        mesh=mesh,
        compiler_params=_compiler_params(),
        scratch_types=[
            pltpu.VMEM((TPW * 128,), jnp.int32),
            pltpu.VMEM((TPW, 8, 128), jnp.float32),
            pltpu.SemaphoreType.DMA,
        ],
    )
    t = run(atomic_numbers)
    return t.transpose(1, 0, 2).reshape(8, N_TILES * 128)[:N_COLS, :N_ATOMS].T


def kernel(atomic_numbers):
    return _embed(atomic_numbers)


# speedup vs baseline: 1.0973x; 1.0246x over previous
import dataclasses
import functools

import numpy as np
import jax
import jax.numpy as jnp
from jax import lax
from jax.experimental import pallas as pl
from jax.experimental.pallas import tpu as pltpu
from jax.experimental.pallas import tpu_sc as plsc

N_ATOMS = 100000
N_COLS = 6
LANES = 16
NUM_WORKERS = 32

N_TILES = 782
TPW = 25
LAST_TPW = N_TILES - 31 * TPW
Z_VALS = (6, 7, 8, 15, 16)


def _compute_tile(a_ref, o_ref, t):
    one = jnp.full((LANES,), 1.0, jnp.float32)
    zero = jnp.zeros((LANES,), jnp.float32)
    for l in range(8):
        a16 = a_ref[pl.ds(t * 128 + l * LANES, LANES)]
        vals = [jnp.where(a16 == z, one, zero) for z in Z_VALS]
        v5 = one - (vals[0] + vals[1] + vals[2] + vals[3] + vals[4])
        vals.append(v5)
        for j in range(N_COLS):
            o_ref[t, j, pl.ds(l * LANES, LANES)] = vals[j]


def _worker_body(a_ref, o_ref, ntiles):
    @pl.loop(0, ntiles)
    def _(t):
        _compute_tile(a_ref, o_ref, t)


SLAB = 5


def _sc_kernel(a_hbm, out_hbm, a_v, o_v, sem):
    wid = lax.axis_index("s") * 2 + lax.axis_index("c")

    @pl.when(wid < 31)
    def _():
        base = wid * (TPW * 128)
        pltpu.sync_copy(a_hbm.at[pl.ds(base, TPW * 128)], a_v)

        @pl.loop(0, TPW // SLAB)
        def _(s):
            @pl.loop(0, SLAB)
            def _(t):
                _compute_tile(a_v, o_v, s * SLAB + t)

            pltpu.async_copy(
                o_v.at[pl.ds(s * SLAB, SLAB)],
                out_hbm.at[pl.ds(wid * TPW + s * SLAB, SLAB)],
                sem,
            )

        pltpu.make_async_copy(out_hbm.at[pl.ds(0, TPW)], o_v, sem).wait()

    @pl.when(wid == 31)
    def _():
        base = 31 * (TPW * 128)
        pltpu.sync_copy(
            a_hbm.at[pl.ds(base, N_ATOMS - base)],
            a_v.at[pl.ds(0, N_ATOMS - base)],
        )
        _worker_body(a_v, o_v, LAST_TPW)
        pltpu.sync_copy(
            o_v.at[pl.ds(0, LAST_TPW)], out_hbm.at[pl.ds(31 * TPW, LAST_TPW)]
        )


def _compiler_params():
    cp = pltpu.CompilerParams()
    if "needs_layout_passes" in pltpu.CompilerParams.__dataclass_fields__:
        cp = dataclasses.replace(cp, needs_layout_passes=False)
    return cp


@jax.jit
def _embed(atomic_numbers):
    mesh = plsc.VectorSubcoreMesh(core_axis_name="c", subcore_axis_name="s")
    run = pl.kernel(
        _sc_kernel,
        out_type=jax.ShapeDtypeStruct((N_TILES, 8, 128), jnp.float32),
        mesh=mesh,
        compiler_params=_compiler_params(),
        scratch_types=[
            pltpu.VMEM((TPW * 128,), jnp.int32),
            pltpu.VMEM((TPW, 8, 128), jnp.float32),
            pltpu.SemaphoreType.DMA,
        ],
    )
    t = run(atomic_numbers)
    return t.transpose(1, 0, 2).reshape(8, N_TILES * 128)[:N_COLS, :N_ATOMS].T


def kernel(atomic_numbers):
    return _embed(atomic_numbers)


# speedup vs baseline: 1.1003x; 1.0027x over previous
import dataclasses

import jax
import jax.numpy as jnp
from jax import lax
from jax.experimental import pallas as pl
from jax.experimental.pallas import tpu as pltpu
from jax.experimental.pallas import tpu_sc as plsc

N_ATOMS = 100000
N_COLS = 6
LANES = 16
NUM_WORKERS = 32

N_TILES = 782
TPW = 25
LAST_TPW = N_TILES - 31 * TPW
Z_VALS = (6, 7, 8, 15, 16)


def _compute_tile(a_ref, o_ref, t):
    one = jnp.full((LANES,), 1.0, jnp.float32)
    zero = jnp.zeros((LANES,), jnp.float32)
    for l in range(8):
        a16 = a_ref[pl.ds(t * 128 + l * LANES, LANES)]
        vals = [jnp.where(a16 == z, one, zero) for z in Z_VALS]
        v5 = one - (vals[0] + vals[1] + vals[2] + vals[3] + vals[4])
        vals.append(v5)
        for j in range(N_COLS):
            o_ref[t, j, pl.ds(l * LANES, LANES)] = vals[j]


def _worker_body(a_ref, o_ref, ntiles):
    @pl.loop(0, ntiles)
    def _(t):
        _compute_tile(a_ref, o_ref, t)


SLAB = 5


def _sc_kernel(a_hbm, out_hbm, a_v, o_v, sem):
    wid = lax.axis_index("s") * 2 + lax.axis_index("c")

    @pl.when(wid < 31)
    def _():
        base = wid * (TPW * 128)
        pltpu.sync_copy(a_hbm.at[pl.ds(base, TPW * 128)], a_v)

        @pl.loop(0, TPW // SLAB)
        def _(s):
            @pl.loop(0, SLAB)
            def _(t):
                _compute_tile(a_v, o_v, s * SLAB + t)

            pltpu.async_copy(
                o_v.at[pl.ds(s * SLAB, SLAB)],
                out_hbm.at[pl.ds(wid * TPW + s * SLAB, SLAB)],
                sem,
            )

        pltpu.make_async_copy(out_hbm.at[pl.ds(0, TPW)], o_v, sem).wait()

    @pl.when(wid == 31)
    def _():
        base = 31 * (TPW * 128)
        pltpu.sync_copy(
            a_hbm.at[pl.ds(base, N_ATOMS - base)],
            a_v.at[pl.ds(0, N_ATOMS - base)],
        )
        _worker_body(a_v, o_v, LAST_TPW)
        pltpu.sync_copy(
            o_v.at[pl.ds(0, LAST_TPW)], out_hbm.at[pl.ds(31 * TPW, LAST_TPW)]
        )


def _compiler_params():
    cp = pltpu.CompilerParams()
    if "needs_layout_passes" in pltpu.CompilerParams.__dataclass_fields__:
        cp = dataclasses.replace(cp, needs_layout_passes=False)
    return cp


@jax.jit
def _embed(atomic_numbers):
    mesh = plsc.VectorSubcoreMesh(core_axis_name="c", subcore_axis_name="s")
    run = pl.kernel(
        _sc_kernel,
        out_type=jax.ShapeDtypeStruct((N_TILES, 8, 128), jnp.float32),
        mesh=mesh,
        compiler_params=_compiler_params(),
        scratch_types=[
            pltpu.VMEM((TPW * 128,), jnp.int32),
            pltpu.VMEM((TPW, 8, 128), jnp.float32),
            pltpu.SemaphoreType.DMA,
        ],
    )
    t = run(atomic_numbers)
    return t.transpose(1, 0, 2).reshape(8, N_TILES * 128)[:N_COLS, :N_ATOMS].T


def kernel(atomic_numbers):
    return _embed(atomic_numbers)


# speedup vs baseline: 1.1552x; 1.0499x over previous
import dataclasses
import functools

import jax
import jax.numpy as jnp
from jax import lax
from jax.experimental import pallas as pl
from jax.experimental.pallas import tpu as pltpu
from jax.experimental.pallas import tpu_sc as plsc

N_ATOMS = 100000
N_COLS = 6
LANES = 16
NUM_WORKERS = 32

N_TILES = 782
TPW = 25
LAST_TPW = N_TILES - 31 * TPW
Z_VALS = (6, 7, 8, 15, 16)


def _compute_tile(a_ref, o_ref, t):
    one = jnp.full((LANES,), 1.0, jnp.float32)
    zero = jnp.zeros((LANES,), jnp.float32)
    for l in range(8):
        a16 = a_ref[pl.ds(t * 128 + l * LANES, LANES)]
        vals = [jnp.where(a16 == z, one, zero) for z in Z_VALS]
        v5 = one - (vals[0] + vals[1] + vals[2] + vals[3] + vals[4])
        vals.append(v5)
        for j in range(N_COLS):
            o_ref[t, j, pl.ds(l * LANES, LANES)] = vals[j]


def _worker_body(a_ref, o_ref, ntiles):
    @pl.loop(0, ntiles)
    def _(t):
        _compute_tile(a_ref, o_ref, t)


SLAB = 5


def _sc_kernel(a_hbm, out_hbm, a_v, o_v, sem):
    wid = lax.axis_index("s") * 2 + lax.axis_index("c")

    @pl.when(wid < 31)
    def _():
        base = wid * (TPW * 128)
        pltpu.sync_copy(a_hbm.at[pl.ds(base, TPW * 128)], a_v)

        @pl.loop(0, TPW // SLAB)
        def _(s):
            @plsc.parallel_loop(0, SLAB)
            def _(t):
                _compute_tile(a_v, o_v, s * SLAB + t)

            pltpu.async_copy(
                o_v.at[pl.ds(s * SLAB, SLAB)],
                out_hbm.at[pl.ds(wid * TPW + s * SLAB, SLAB)],
                sem,
            )

        pltpu.make_async_copy(out_hbm.at[pl.ds(0, TPW)], o_v, sem).wait()

    @pl.when(wid == 31)
    def _():
        base = 31 * (TPW * 128)
        pltpu.sync_copy(
            a_hbm.at[pl.ds(base, N_ATOMS - base)],
            a_v.at[pl.ds(0, N_ATOMS - base)],
        )
        _worker_body(a_v, o_v, LAST_TPW)
        pltpu.sync_copy(
            o_v.at[pl.ds(0, LAST_TPW)], out_hbm.at[pl.ds(31 * TPW, LAST_TPW)]
        )


def _compiler_params():
    cp = pltpu.CompilerParams()
    if "needs_layout_passes" in pltpu.CompilerParams.__dataclass_fields__:
        cp = dataclasses.replace(cp, needs_layout_passes=False)
    return cp


@jax.jit
def _embed(atomic_numbers):
    mesh = plsc.VectorSubcoreMesh(core_axis_name="c", subcore_axis_name="s")
    run = pl.kernel(
        _sc_kernel,
        out_type=jax.ShapeDtypeStruct((N_TILES, 8, 128), jnp.float32),
        mesh=mesh,
        compiler_params=_compiler_params(),
        scratch_types=[
            pltpu.VMEM((TPW * 128,), jnp.int32),
            pltpu.VMEM((TPW, 8, 128), jnp.float32),
            pltpu.SemaphoreType.DMA,
        ],
    )
    t = run(atomic_numbers)
    return t.transpose(1, 0, 2).reshape(8, N_TILES * 128)[:N_COLS, :N_ATOMS].T


def kernel(atomic_numbers):
    return _embed(atomic_numbers)
